# baseline (speedup 1.0000x reference)
import functools

import jax
import jax.numpy as jnp
from jax import lax
from jax.experimental import pallas as pl
from jax.experimental.pallas import tpu as pltpu
from jax.experimental.pallas import tpu_sc as plsc

N = 10000
E = 320000
D_IN = 128
H = 256
OUT = 10
G = 64
EPS = 1e-5

NPAD = 10112
CHUNK = 64
NBUF = 4
EPAD = 327680
ZROWS = NPAD // 16

_f32 = jnp.float32


def _seg_body(edge_split, gather, *refs):
    if gather:
        (tcat, src_hbm, dst_hbm, zs_hbm, s_out,
         srcb, dstb, gbuf, s_sh) = refs[:9]
        sems = refs[9:]
    else:
        (ones_hbm, dst_hbm, zs_hbm, s_out,
         srcb, dstb, gbuf, s_sh) = refs[:8]
        sems = refs[8:]
    gsem = sems[:NBUF]
    ssem = sems[NBUF:]
    c = lax.axis_index("c")
    s = lax.axis_index("s")
    nchunks = EPAD // CHUNK // (32 if edge_split else 16)
    tile = c * 16 + s if edge_split else s
    e0 = tile * nchunks * CHUNK
    se0 = c * EPAD + e0

    z0 = s * ZROWS
    pltpu.sync_copy(zs_hbm.at[pl.ds(z0, ZROWS)], s_sh.at[pl.ds(z0, ZROWS)])
    if not gather:
        for b in range(NBUF):
            pltpu.sync_copy(ones_hbm, gbuf.at[b])
    plsc.subcore_barrier()

    def stage(j, b):
        pltpu.sync_copy(dst_hbm.at[pl.ds(e0 + j * CHUNK, CHUNK)], dstb.at[b])
        if gather:
            pltpu.sync_copy(src_hbm.at[pl.ds(se0 + j * CHUNK, CHUNK)],
                            srcb.at[b])

    def fire_gather(b):
        if gather:
            pltpu.async_copy(tcat.at[srcb.at[b]], gbuf.at[b], gsem[b])

    def wait_gather(b):
        if gather:
            pltpu.make_async_copy(tcat.at[srcb.at[b]], gbuf.at[b],
                                  gsem[b]).wait()

    def fire_scatter(b):
        pltpu.async_copy(gbuf.at[b], s_sh.at[dstb.at[b]], ssem[b], add=True)

    def wait_scatter(b):
        pltpu.make_async_copy(gbuf.at[b], s_sh.at[dstb.at[b]], ssem[b]).wait()

    for b in range(NBUF):
        stage(b, b)
        fire_gather(b)

    def _step(jn, carry):
        j = jn * NBUF
        for b in range(NBUF):
            wait_gather(b)
            fire_scatter(b)
        for b in range(NBUF):
            wait_scatter(b)
            stage(j + NBUF + b, b)
            fire_gather(b)
        return carry

    lax.fori_loop(0, nchunks // NBUF - 1, _step, 0)

    for b in range(NBUF):
        wait_gather(b)
        fire_scatter(b)
    for b in range(NBUF):
        wait_scatter(b)

    plsc.subcore_barrier()

    o0 = c * NPAD + z0
    pltpu.sync_copy(s_sh.at[pl.ds(z0, ZROWS)], s_out.at[pl.ds(o0, ZROWS)])


def _make_seg(edge_split, gather=True):
    return pl.kernel(
        functools.partial(_seg_body, edge_split, gather),
        out_type=[jax.ShapeDtypeStruct((2 * NPAD, 128), _f32)],
        mesh=plsc.VectorSubcoreMesh(core_axis_name="c", subcore_axis_name="s",
                                    num_cores=2, num_subcores=16),
        scratch_types=[
            pltpu.VMEM((NBUF, CHUNK), jnp.int32),
            pltpu.VMEM((NBUF, CHUNK), jnp.int32),
            pltpu.VMEM((NBUF, CHUNK, 128), _f32),
            pltpu.VMEM_SHARED((NPAD, 128), _f32),
        ] + [pltpu.SemaphoreType.DMA] * (2 * NBUF),
    )


def _norm_body(x_ref, o_ref):
    x = x_ref[...]
    c0 = x[:, 0:1]
    c1 = x[:, 1:2]
    mn0, mx0 = jnp.min(c0), jnp.max(c0)
    mn1, mx1 = jnp.min(c1), jnp.max(c1)
    col = lax.broadcasted_iota(jnp.int32, (1, D_IN), 1)
    scale = jnp.where(col == 0, 1.0 / mx0, jnp.where(col == 1, 1.0 / mx1, 1.0))
    shift = jnp.where(col == 0, -mn0 / mx0, jnp.where(col == 1, -mn1 / mx1, 0.0))
    o_ref[...] = x * scale + shift


def _norm_x(x):
    return pl.pallas_call(
        _norm_body,
        out_shape=jax.ShapeDtypeStruct((N, D_IN), _f32),
    )(x)


_BN_SCALE = 1.0 / (1.0 + EPS) ** 0.5
_RBLK = 1000
_GRID = N // _RBLK


def _row_spec(w):
    return pl.BlockSpec((_RBLK, w), lambda i: (i, 0))


def _full_spec(h, w):
    return pl.BlockSpec((h, w), lambda i: (0, 0))


def _pre1_body(xn_ref, wr_ref, b_ref, r_ref):
    r_ref[...] = (jnp.dot(xn_ref[...], wr_ref[...],
                          preferred_element_type=_f32) + b_ref[...])


def _pre1(xn, wr, b):
    return pl.pallas_call(
        _pre1_body,
        grid=(_GRID,),
        in_specs=[_row_spec(128), _full_spec(128, 256), _full_spec(1, 256)],
        out_specs=_row_spec(256),
        out_shape=jax.ShapeDtypeStruct((N, 256), _f32),
    )(xn, wr, b)


def _pre23_body(h0_ref, h1_ref, wra_ref, wrb_ref, b_ref, r_ref):
    r_ref[...] = (jnp.dot(h0_ref[...], wra_ref[...],
                          preferred_element_type=_f32)
                  + jnp.dot(h1_ref[...], wrb_ref[...],
                            preferred_element_type=_f32)
                  + b_ref[...])


def _pre23(h0, h1, wra, wrb, b):
    return pl.pallas_call(
        _pre23_body,
        grid=(_GRID,),
        in_specs=[_row_spec(128), _row_spec(128),
                  _full_spec(128, 256), _full_spec(128, 256),
                  _full_spec(1, 256)],
        out_specs=_row_spec(256),
        out_shape=jax.ShapeDtypeStruct((N, 256), _f32),
    )(h0, h1, wra, wrb, b)


def _post1_body(xn_ref, s0_ref, s1_ref, c0_ref, c1_ref, r_ref, wl_ref,
                g_ref, be_ref, h0_ref, h1_ref):
    deg = c0_ref[:, 0:1] + c1_ref[:, 0:1]
    inv = 1.0 / (deg + 1.0)
    mean = (s0_ref[...] + s1_ref[...] + xn_ref[...]) * inv
    y = jnp.dot(mean, wl_ref[...], preferred_element_type=_f32) + r_ref[...]
    y = y * (g_ref[...] * _BN_SCALE) + be_ref[...]
    y = jnp.maximum(y, 0.0)
    h0_ref[...] = y[:, :128]
    h1_ref[...] = y[:, 128:]


def _post1(xn, s0, s1, c0, c1, r, wl, g, be):
    return pl.pallas_call(
        _post1_body,
        grid=(_GRID,),
        in_specs=[_row_spec(128), _row_spec(128), _row_spec(128),
                  _row_spec(16), _row_spec(16), _row_spec(256),
                  _full_spec(128, 256), _full_spec(1, 256),
                  _full_spec(1, 256)],
        out_specs=[_row_spec(128), _row_spec(128)],
        out_shape=[jax.ShapeDtypeStruct((N, 128), _f32)] * 2,
    )(xn, s0, s1, c0, c1, r, wl, g, be)


def _post23_body(s0_ref, s1_ref, h0_ref, h1_ref, c0_ref, c1_ref, r_ref,
                 wla_ref, wlb_ref, g_ref, be_ref, o0_ref, o1_ref):
    deg = c0_ref[:, 0:1] + c1_ref[:, 0:1]
    inv = 1.0 / (deg + 1.0)
    m0 = (s0_ref[...] + h0_ref[...]) * inv
    m1 = (s1_ref[...] + h1_ref[...]) * inv
    y = (jnp.dot(m0, wla_ref[...], preferred_element_type=_f32)
         + jnp.dot(m1, wlb_ref[...], preferred_element_type=_f32)
         + r_ref[...])
    y = y * (g_ref[...] * _BN_SCALE) + be_ref[...]
    y = jnp.maximum(y, 0.0)
    o0_ref[...] = y[:, :128]
    o1_ref[...] = y[:, 128:]


def _post23(s0, s1, h0, h1, c0, c1, r, wla, wlb, g, be):
    return pl.pallas_call(
        _post23_body,
        grid=(_GRID,),
        in_specs=[_row_spec(128)] * 4 + [_row_spec(16)] * 2 +
                 [_row_spec(256)] +
                 [_full_spec(128, 256)] * 2 + [_full_spec(1, 256)] * 2,
        out_specs=[_row_spec(128), _row_spec(128)],
        out_shape=[jax.ShapeDtypeStruct((N, 128), _f32)] * 2,
    )(s0, s1, h0, h1, c0, c1, r, wla, wlb, g, be)


def _pool_body(h0_ref, h1_ref, batch_ref, wf1_ref, bf1_ref, wf2_ref, bf2_ref,
               o_ref):
    ids = lax.broadcasted_iota(jnp.int32, (G, N), 0)
    oh = (ids == batch_ref[...]).astype(_f32)
    gc = jnp.sum(oh, axis=1, keepdims=True)
    g0 = jnp.dot(oh, h0_ref[...], preferred_element_type=_f32)
    g1 = jnp.dot(oh, h1_ref[...], preferred_element_type=_f32)
    hg = jnp.concatenate([g0, g1], axis=1) / jnp.maximum(gc, 1.0)
    z1 = jnp.maximum(jnp.dot(hg, wf1_ref[...], preferred_element_type=_f32)
                     + bf1_ref[...], 0.0)
    z = jnp.dot(z1, wf2_ref[...], preferred_element_type=_f32) + bf2_ref[...]
    col = lax.broadcasted_iota(jnp.int32, (G, 128), 1)
    z = jnp.where(col < OUT, z, -jnp.inf)
    m = jnp.max(z, axis=1, keepdims=True)
    lse = jnp.log(jnp.sum(jnp.exp(z - m), axis=1, keepdims=True)) + m
    o_ref[...] = z - lse


def _pool(h0, h1, batch2d, wf1, bf1, wf2p, bf2p):
    return pl.pallas_call(
        _pool_body,
        out_shape=jax.ShapeDtypeStruct((G, 128), _f32),
    )(h0, h1, batch2d, wf1, bf1, wf2p, bf2p)


def kernel(x, edge_index, batch, Wl1, Wr1, b1, g1, be1, Wl2, Wr2, b2, g2, be2,
           Wl3, Wr3, b3, g3, be3, Wf1, bf1, Wf2, bf2):
    pad = EPAD - E
    src_f = jnp.concatenate([edge_index[0], jnp.zeros((pad,), jnp.int32)])
    src2 = jnp.concatenate([src_f, src_f + N])
    dst_f = jnp.concatenate([edge_index[1], jnp.full((pad,), N, jnp.int32)])
    zs = jnp.zeros((NPAD, 128), _f32)
    ones = jnp.ones((CHUNK, 128), _f32)

    xn = _norm_x(x)

    cntk = _make_seg(edge_split=True, gather=False)
    (cntp,) = cntk(ones, dst_f, zs)
    c0, c1 = cntp[:N, :16], cntp[NPAD:NPAD + N, :16]

    seg1 = _make_seg(edge_split=True)
    tcat = jnp.concatenate([xn, xn], axis=0)
    r1 = _pre1(xn, Wr1, b1.reshape(1, H))
    (sp,) = seg1(tcat, src2, dst_f, zs)
    h0, h1 = _post1(xn, sp[:N], sp[NPAD:NPAD + N], c0, c1, r1, Wl1,
                    g1.reshape(1, H), be1.reshape(1, H))

    seg = _make_seg(edge_split=False)

    r2 = _pre23(h0, h1, Wr2[:128], Wr2[128:], b2.reshape(1, H))
    (sp,) = seg(jnp.concatenate([h0, h1], axis=0), src2, dst_f, zs)
    h0, h1 = _post23(sp[:N], sp[NPAD:NPAD + N], h0, h1, c0, c1, r2,
                     Wl2[:128], Wl2[128:], g2.reshape(1, H),
                     be2.reshape(1, H))

    r3 = _pre23(h0, h1, Wr3[:128], Wr3[128:], b3.reshape(1, H))
    (sp,) = seg(jnp.concatenate([h0, h1], axis=0), src2, dst_f, zs)
    h0, h1 = _post23(sp[:N], sp[NPAD:NPAD + N], h0, h1, c0, c1, r3,
                     Wl3[:128], Wl3[128:], g3.reshape(1, H),
                     be3.reshape(1, H))

    wf2p = jnp.pad(Wf2, ((0, 0), (0, 128 - OUT)))
    bf2p = jnp.pad(bf2, (0, 128 - OUT)).reshape(1, 128)
    out = _pool(h0, h1, batch.reshape(1, N), Wf1, bf1.reshape(1, H // 2),
                wf2p, bf2p)
    return out[:, :OUT]


# speedup vs baseline: 1.5904x; 1.5904x over previous
import functools

import jax
import jax.numpy as jnp
from jax import lax
from jax.experimental import pallas as pl
from jax.experimental.pallas import tpu as pltpu
from jax.experimental.pallas import tpu_sc as plsc

N = 10000
E = 320000
D_IN = 128
H = 256
OUT = 10
G = 64
EPS = 1e-5

NPAD = 10112
CHUNK = 112
NBUF = 3
EPAD = 322560
ZROWS = NPAD // 16

_f32 = jnp.float32


def _seg_body(edge_split, gather, *refs):
    if gather:
        (tcat, src_hbm, dst_hbm, zs_hbm, s_out,
         srcb, dstb, gbuf, s_sh) = refs[:9]
        sems = refs[9:]
    else:
        (ones_hbm, dst_hbm, zs_hbm, s_out,
         srcb, dstb, gbuf, s_sh) = refs[:8]
        sems = refs[8:]
    gsem = sems[:NBUF]
    ssem = sems[NBUF:]
    c = lax.axis_index("c")
    s = lax.axis_index("s")
    nchunks = EPAD // CHUNK // (32 if edge_split else 16)
    tile = c * 16 + s if edge_split else s
    e0 = tile * nchunks * CHUNK
    se0 = c * EPAD + e0

    z0 = s * ZROWS
    pltpu.sync_copy(zs_hbm.at[pl.ds(z0, ZROWS)], s_sh.at[pl.ds(z0, ZROWS)])
    if not gather:
        for b in range(NBUF):
            pltpu.sync_copy(ones_hbm, gbuf.at[b])
    plsc.subcore_barrier()

    def stage(j, b):
        pltpu.sync_copy(dst_hbm.at[pl.ds(e0 + j * CHUNK, CHUNK)], dstb.at[b])
        if gather:
            pltpu.sync_copy(src_hbm.at[pl.ds(se0 + j * CHUNK, CHUNK)],
                            srcb.at[b])

    def fire_gather(b):
        if gather:
            pltpu.async_copy(tcat.at[srcb.at[b]], gbuf.at[b], gsem[b])

    def wait_gather(b):
        if gather:
            pltpu.make_async_copy(tcat.at[srcb.at[b]], gbuf.at[b],
                                  gsem[b]).wait()

    def fire_scatter(b):
        pltpu.async_copy(gbuf.at[b], s_sh.at[dstb.at[b]], ssem[b], add=True)

    def wait_scatter(b):
        pltpu.make_async_copy(gbuf.at[b], s_sh.at[dstb.at[b]], ssem[b]).wait()

    for b in range(NBUF):
        stage(b, b)
        fire_gather(b)

    def _step(jn, carry):
        j = jn * NBUF
        for b in range(NBUF):
            wait_gather(b)
            fire_scatter(b)
        for b in range(NBUF):
            wait_scatter(b)
            stage(j + NBUF + b, b)
            fire_gather(b)
        return carry

    lax.fori_loop(0, nchunks // NBUF - 1, _step, 0)

    for b in range(NBUF):
        wait_gather(b)
        fire_scatter(b)
    for b in range(NBUF):
        wait_scatter(b)

    plsc.subcore_barrier()

    o0 = c * NPAD + z0
    pltpu.sync_copy(s_sh.at[pl.ds(z0, ZROWS)], s_out.at[pl.ds(o0, ZROWS)])


def _make_seg(edge_split, gather=True):
    return pl.kernel(
        functools.partial(_seg_body, edge_split, gather),
        out_type=[jax.ShapeDtypeStruct((2 * NPAD, 128), _f32)],
        mesh=plsc.VectorSubcoreMesh(core_axis_name="c", subcore_axis_name="s",
                                    num_cores=2, num_subcores=16),
        scratch_types=[
            pltpu.VMEM((NBUF, CHUNK), jnp.int32),
            pltpu.VMEM((NBUF, CHUNK), jnp.int32),
            pltpu.VMEM((NBUF, CHUNK, 128), _f32),
            pltpu.VMEM_SHARED((NPAD, 128), _f32),
        ] + [pltpu.SemaphoreType.DMA] * (2 * NBUF),
    )


def _norm_body(x_ref, o_ref):
    x = x_ref[...]
    c0 = x[:, 0:1]
    c1 = x[:, 1:2]
    mn0, mx0 = jnp.min(c0), jnp.max(c0)
    mn1, mx1 = jnp.min(c1), jnp.max(c1)
    col = lax.broadcasted_iota(jnp.int32, (1, D_IN), 1)
    scale = jnp.where(col == 0, 1.0 / mx0, jnp.where(col == 1, 1.0 / mx1, 1.0))
    shift = jnp.where(col == 0, -mn0 / mx0, jnp.where(col == 1, -mn1 / mx1, 0.0))
    o_ref[...] = x * scale + shift


def _norm_x(x):
    return pl.pallas_call(
        _norm_body,
        out_shape=jax.ShapeDtypeStruct((N, D_IN), _f32),
    )(x)


_BN_SCALE = 1.0 / (1.0 + EPS) ** 0.5
_RBLK = 1000
_GRID = N // _RBLK


def _row_spec(w):
    return pl.BlockSpec((_RBLK, w), lambda i: (i, 0))


def _full_spec(h, w):
    return pl.BlockSpec((h, w), lambda i: (0, 0))


def _pre1_body(xn_ref, wr_ref, b_ref, r_ref):
    r_ref[...] = (jnp.dot(xn_ref[...], wr_ref[...],
                          preferred_element_type=_f32) + b_ref[...])


def _pre1(xn, wr, b):
    return pl.pallas_call(
        _pre1_body,
        grid=(_GRID,),
        in_specs=[_row_spec(128), _full_spec(128, 256), _full_spec(1, 256)],
        out_specs=_row_spec(256),
        out_shape=jax.ShapeDtypeStruct((N, 256), _f32),
    )(xn, wr, b)


def _pre23_body(h0_ref, h1_ref, wra_ref, wrb_ref, b_ref, r_ref):
    r_ref[...] = (jnp.dot(h0_ref[...], wra_ref[...],
                          preferred_element_type=_f32)
                  + jnp.dot(h1_ref[...], wrb_ref[...],
                            preferred_element_type=_f32)
                  + b_ref[...])


def _pre23(h0, h1, wra, wrb, b):
    return pl.pallas_call(
        _pre23_body,
        grid=(_GRID,),
        in_specs=[_row_spec(128), _row_spec(128),
                  _full_spec(128, 256), _full_spec(128, 256),
                  _full_spec(1, 256)],
        out_specs=_row_spec(256),
        out_shape=jax.ShapeDtypeStruct((N, 256), _f32),
    )(h0, h1, wra, wrb, b)


def _post1_body(xn_ref, s0_ref, s1_ref, c0_ref, c1_ref, r_ref, wl_ref,
                g_ref, be_ref, h0_ref, h1_ref):
    deg = c0_ref[:, 0:1] + c1_ref[:, 0:1]
    inv = 1.0 / (deg + 1.0)
    mean = (s0_ref[...] + s1_ref[...] + xn_ref[...]) * inv
    y = jnp.dot(mean, wl_ref[...], preferred_element_type=_f32) + r_ref[...]
    y = y * (g_ref[...] * _BN_SCALE) + be_ref[...]
    y = jnp.maximum(y, 0.0)
    h0_ref[...] = y[:, :128]
    h1_ref[...] = y[:, 128:]


def _post1(xn, s0, s1, c0, c1, r, wl, g, be):
    return pl.pallas_call(
        _post1_body,
        grid=(_GRID,),
        in_specs=[_row_spec(128), _row_spec(128), _row_spec(128),
                  _row_spec(16), _row_spec(16), _row_spec(256),
                  _full_spec(128, 256), _full_spec(1, 256),
                  _full_spec(1, 256)],
        out_specs=[_row_spec(128), _row_spec(128)],
        out_shape=[jax.ShapeDtypeStruct((N, 128), _f32)] * 2,
    )(xn, s0, s1, c0, c1, r, wl, g, be)


def _post23_body(s0_ref, s1_ref, h0_ref, h1_ref, c0_ref, c1_ref, r_ref,
                 wla_ref, wlb_ref, g_ref, be_ref, o0_ref, o1_ref):
    deg = c0_ref[:, 0:1] + c1_ref[:, 0:1]
    inv = 1.0 / (deg + 1.0)
    m0 = (s0_ref[...] + h0_ref[...]) * inv
    m1 = (s1_ref[...] + h1_ref[...]) * inv
    y = (jnp.dot(m0, wla_ref[...], preferred_element_type=_f32)
         + jnp.dot(m1, wlb_ref[...], preferred_element_type=_f32)
         + r_ref[...])
    y = y * (g_ref[...] * _BN_SCALE) + be_ref[...]
    y = jnp.maximum(y, 0.0)
    o0_ref[...] = y[:, :128]
    o1_ref[...] = y[:, 128:]


def _post23(s0, s1, h0, h1, c0, c1, r, wla, wlb, g, be):
    return pl.pallas_call(
        _post23_body,
        grid=(_GRID,),
        in_specs=[_row_spec(128)] * 4 + [_row_spec(16)] * 2 +
                 [_row_spec(256)] +
                 [_full_spec(128, 256)] * 2 + [_full_spec(1, 256)] * 2,
        out_specs=[_row_spec(128), _row_spec(128)],
        out_shape=[jax.ShapeDtypeStruct((N, 128), _f32)] * 2,
    )(s0, s1, h0, h1, c0, c1, r, wla, wlb, g, be)


def _pool_body(h0_ref, h1_ref, batch_ref, wf1_ref, bf1_ref, wf2_ref, bf2_ref,
               o_ref):
    ids = lax.broadcasted_iota(jnp.int32, (G, N), 0)
    oh = (ids == batch_ref[...]).astype(_f32)
    gc = jnp.sum(oh, axis=1, keepdims=True)
    g0 = jnp.dot(oh, h0_ref[...], preferred_element_type=_f32)
    g1 = jnp.dot(oh, h1_ref[...], preferred_element_type=_f32)
    hg = jnp.concatenate([g0, g1], axis=1) / jnp.maximum(gc, 1.0)
    z1 = jnp.maximum(jnp.dot(hg, wf1_ref[...], preferred_element_type=_f32)
                     + bf1_ref[...], 0.0)
    z = jnp.dot(z1, wf2_ref[...], preferred_element_type=_f32) + bf2_ref[...]
    col = lax.broadcasted_iota(jnp.int32, (G, 128), 1)
    z = jnp.where(col < OUT, z, -jnp.inf)
    m = jnp.max(z, axis=1, keepdims=True)
    lse = jnp.log(jnp.sum(jnp.exp(z - m), axis=1, keepdims=True)) + m
    o_ref[...] = z - lse


def _pool(h0, h1, batch2d, wf1, bf1, wf2p, bf2p):
    return pl.pallas_call(
        _pool_body,
        out_shape=jax.ShapeDtypeStruct((G, 128), _f32),
    )(h0, h1, batch2d, wf1, bf1, wf2p, bf2p)


def kernel(x, edge_index, batch, Wl1, Wr1, b1, g1, be1, Wl2, Wr2, b2, g2, be2,
           Wl3, Wr3, b3, g3, be3, Wf1, bf1, Wf2, bf2):
    pad = EPAD - E
    src_f = jnp.concatenate([edge_index[0], jnp.zeros((pad,), jnp.int32)])
    src2 = jnp.concatenate([src_f, src_f + N])
    dst_f = jnp.concatenate([edge_index[1], jnp.full((pad,), N, jnp.int32)])
    zs = jnp.zeros((NPAD, 128), _f32)
    ones = jnp.ones((CHUNK, 128), _f32)

    xn = _norm_x(x)

    cntk = _make_seg(edge_split=True, gather=False)
    (cntp,) = cntk(ones, dst_f, zs)
    c0, c1 = cntp[:N, :16], cntp[NPAD:NPAD + N, :16]

    seg1 = _make_seg(edge_split=True)
    tcat = jnp.concatenate([xn, xn], axis=0)
    r1 = _pre1(xn, Wr1, b1.reshape(1, H))
    (sp,) = seg1(tcat, src2, dst_f, zs)
    h0, h1 = _post1(xn, sp[:N], sp[NPAD:NPAD + N], c0, c1, r1, Wl1,
                    g1.reshape(1, H), be1.reshape(1, H))

    seg = _make_seg(edge_split=False)

    r2 = _pre23(h0, h1, Wr2[:128], Wr2[128:], b2.reshape(1, H))
    (sp,) = seg(jnp.concatenate([h0, h1], axis=0), src2, dst_f, zs)
    h0, h1 = _post23(sp[:N], sp[NPAD:NPAD + N], h0, h1, c0, c1, r2,
                     Wl2[:128], Wl2[128:], g2.reshape(1, H),
                     be2.reshape(1, H))

    r3 = _pre23(h0, h1, Wr3[:128], Wr3[128:], b3.reshape(1, H))
    (sp,) = seg(jnp.concatenate([h0, h1], axis=0), src2, dst_f, zs)
    h0, h1 = _post23(sp[:N], sp[NPAD:NPAD + N], h0, h1, c0, c1, r3,
                     Wl3[:128], Wl3[128:], g3.reshape(1, H),
                     be3.reshape(1, H))

    wf2p = jnp.pad(Wf2, ((0, 0), (0, 128 - OUT)))
    bf2p = jnp.pad(bf2, (0, 128 - OUT)).reshape(1, 128)
    out = _pool(h0, h1, batch.reshape(1, N), Wf1, bf1.reshape(1, H // 2),
                wf2p, bf2p)
    return out[:, :OUT]
